# baseline (speedup 1.0000x reference)
import functools

import jax
import jax.numpy as jnp
from jax.experimental import pallas as pl
from jax.experimental.pallas import tpu as pltpu

_LANE = 128


def _round_up(v, m):
    return ((v + m - 1) // m) * m


def _fused_kernel(a_ref, x_ref, w_ref, b_ref, o_ref):
    a16 = a_ref[...].astype(jnp.bfloat16)
    s = jnp.dot(a16, x_ref[...], preferred_element_type=jnp.float32)
    o_ref[...] = (jnp.dot(s.astype(jnp.bfloat16), w_ref[...],
                          preferred_element_type=jnp.float32)
                  + b_ref[...])


def _pad2d(arr, rows, cols):
    r, c = arr.shape
    if r == rows and c == cols:
        return arr
    return jnp.pad(arr, ((0, rows - r), (0, cols - c)))


@jax.jit
def _gcn_forward(x, adj, w, b):
    n, f_in = x.shape
    f_out = w.shape[1]

    n_p = _round_up(n, _LANE)
    fin_p = _round_up(f_in, _LANE)
    fout_p = _round_up(f_out, _LANE)

    tm = 256
    while n_p % tm:
        tm //= 2

    adj_p = _pad2d(adj, n_p, n_p)
    x_p = _pad2d(x.astype(jnp.bfloat16), n_p, fin_p)
    w_p = _pad2d(w.astype(jnp.bfloat16), fin_p, fout_p)
    b_p = jnp.pad(b.astype(jnp.float32), (0, fout_p - f_out)).reshape(1, fout_p)

    cost = pl.CostEstimate(
        flops=2 * n_p * n_p * fin_p + 2 * n_p * fin_p * fout_p,
        transcendentals=0,
        bytes_accessed=(4 * n_p * n_p
                        + 2 * n_p * fin_p
                        + 2 * fin_p * fout_p
                        + 4 * fout_p
                        + 4 * n_p * fout_p))

    out = pl.pallas_call(
        _fused_kernel,
        out_shape=jax.ShapeDtypeStruct((n_p, fout_p), jnp.float32),
        grid=(n_p // tm,),
        in_specs=[
            pl.BlockSpec((tm, n_p), lambda i: (i, 0)),
            pl.BlockSpec((n_p, fin_p), lambda i: (0, 0)),
            pl.BlockSpec((fin_p, fout_p), lambda i: (0, 0)),
            pl.BlockSpec((1, fout_p), lambda i: (0, 0)),
        ],
        out_specs=pl.BlockSpec((tm, fout_p), lambda i: (i, 0)),
        compiler_params=pltpu.CompilerParams(
            dimension_semantics=("parallel",),
            vmem_limit_bytes=56 * 1024 * 1024),
        cost_estimate=cost,
    )(adj_p, x_p, w_p, b_p)

    if n_p == n and fout_p == f_out:
        return out
    return out[:n, :f_out]


def kernel(x, adj, w, b):
    return _gcn_forward(x, adj, w, b)


# speedup vs baseline: 1.1426x; 1.1426x over previous
import functools

import jax
import jax.numpy as jnp
from jax.experimental import pallas as pl
from jax.experimental.pallas import tpu as pltpu

_LANE = 128


def _round_up(v, m):
    return ((v + m - 1) // m) * m


def _fused_kernel(a_ref, x_ref, w_ref, b_ref, o_ref):
    a16 = a_ref[...].astype(jnp.bfloat16)
    s = jnp.dot(a16, x_ref[...], preferred_element_type=jnp.float32)
    o_ref[...] = (jnp.dot(s.astype(jnp.bfloat16), w_ref[...],
                          preferred_element_type=jnp.float32)
                  + b_ref[...])


def _pad2d(arr, rows, cols):
    r, c = arr.shape
    if r == rows and c == cols:
        return arr
    return jnp.pad(arr, ((0, rows - r), (0, cols - c)))


@jax.jit
def _gcn_forward(x, adj, w, b):
    n, f_in = x.shape
    f_out = w.shape[1]

    n_p = _round_up(n, _LANE)
    fin_p = _round_up(f_in, _LANE)
    fout_p = _round_up(f_out, _LANE)

    tm = 1024
    while n_p % tm:
        tm //= 2

    adj_p = _pad2d(adj, n_p, n_p)
    x_p = _pad2d(x.astype(jnp.bfloat16), n_p, fin_p)
    w_p = _pad2d(w.astype(jnp.bfloat16), fin_p, fout_p)
    b_p = jnp.pad(b.astype(jnp.float32), (0, fout_p - f_out)).reshape(1, fout_p)

    cost = pl.CostEstimate(
        flops=2 * n_p * n_p * fin_p + 2 * n_p * fin_p * fout_p,
        transcendentals=0,
        bytes_accessed=(4 * n_p * n_p
                        + 2 * n_p * fin_p
                        + 2 * fin_p * fout_p
                        + 4 * fout_p
                        + 4 * n_p * fout_p))

    out = pl.pallas_call(
        _fused_kernel,
        out_shape=jax.ShapeDtypeStruct((n_p, fout_p), jnp.float32),
        grid=(n_p // tm,),
        in_specs=[
            pl.BlockSpec((tm, n_p), lambda i: (i, 0)),
            pl.BlockSpec((n_p, fin_p), lambda i: (0, 0)),
            pl.BlockSpec((fin_p, fout_p), lambda i: (0, 0)),
            pl.BlockSpec((1, fout_p), lambda i: (0, 0)),
        ],
        out_specs=pl.BlockSpec((tm, fout_p), lambda i: (i, 0)),
        compiler_params=pltpu.CompilerParams(
            dimension_semantics=("parallel",),
            vmem_limit_bytes=56 * 1024 * 1024),
        cost_estimate=cost,
    )(adj_p, x_p, w_p, b_p)

    if n_p == n and fout_p == f_out:
        return out
    return out[:n, :f_out]


def kernel(x, adj, w, b):
    return _gcn_forward(x, adj, w, b)


# speedup vs baseline: 1.3477x; 1.1795x over previous
import functools

import jax
import jax.numpy as jnp
from jax.experimental import pallas as pl
from jax.experimental.pallas import tpu as pltpu

_LANE = 128


def _round_up(v, m):
    return ((v + m - 1) // m) * m


def _fused_kernel(a_ref, x_ref, w_ref, b_ref, o_ref):
    a16 = a_ref[...].astype(jnp.bfloat16)
    x16 = x_ref[...].astype(jnp.bfloat16)
    s = jnp.dot(a16, x16, preferred_element_type=jnp.float32)
    o_ref[...] = (jnp.dot(s.astype(jnp.bfloat16),
                          w_ref[...].astype(jnp.bfloat16),
                          preferred_element_type=jnp.float32)
                  + b_ref[...])


def _pad2d(arr, rows, cols):
    r, c = arr.shape
    if r == rows and c == cols:
        return arr
    return jnp.pad(arr, ((0, rows - r), (0, cols - c)))


@jax.jit
def _gcn_forward(x, adj, w, b):
    n, f_in = x.shape
    f_out = w.shape[1]

    n_p = _round_up(n, _LANE)
    fin_p = _round_up(f_in, _LANE)
    fout_p = _round_up(f_out, _LANE)

    tm = 512
    while n_p % tm:
        tm //= 2

    adj_p = _pad2d(adj, n_p, n_p)
    x_p = _pad2d(x, n_p, fin_p)
    w_p = _pad2d(w, fin_p, fout_p)
    b_p = jnp.pad(b.astype(jnp.float32), (0, fout_p - f_out)).reshape(1, fout_p)

    cost = pl.CostEstimate(
        flops=2 * n_p * n_p * fin_p + 2 * n_p * fin_p * fout_p,
        transcendentals=0,
        bytes_accessed=(4 * n_p * n_p
                        + 2 * n_p * fin_p
                        + 2 * fin_p * fout_p
                        + 4 * fout_p
                        + 4 * n_p * fout_p))

    out = pl.pallas_call(
        _fused_kernel,
        out_shape=jax.ShapeDtypeStruct((n_p, fout_p), jnp.float32),
        grid=(n_p // tm,),
        in_specs=[
            pl.BlockSpec((tm, n_p), lambda i: (i, 0)),
            pl.BlockSpec((n_p, fin_p), lambda i: (0, 0)),
            pl.BlockSpec((fin_p, fout_p), lambda i: (0, 0)),
            pl.BlockSpec((1, fout_p), lambda i: (0, 0)),
        ],
        out_specs=pl.BlockSpec((tm, fout_p), lambda i: (i, 0)),
        compiler_params=pltpu.CompilerParams(
            dimension_semantics=("parallel",),
            vmem_limit_bytes=56 * 1024 * 1024),
        cost_estimate=cost,
    )(adj_p, x_p, w_p, b_p)

    if n_p == n and fout_p == f_out:
        return out
    return out[:n, :f_out]


def kernel(x, adj, w, b):
    return _gcn_forward(x, adj, w, b)


# speedup vs baseline: 1.3498x; 1.0016x over previous
import functools

import jax
import jax.numpy as jnp
from jax.experimental import pallas as pl
from jax.experimental.pallas import tpu as pltpu

_LANE = 128


def _round_up(v, m):
    return ((v + m - 1) // m) * m


def _fused_kernel(a_ref, x_ref, w_ref, b_ref, o_ref):
    a16 = a_ref[...].astype(jnp.bfloat16)
    x16 = x_ref[...].astype(jnp.bfloat16)
    s = jnp.dot(a16, x16, preferred_element_type=jnp.float32)
    o_ref[...] = (jnp.dot(s.astype(jnp.bfloat16),
                          w_ref[...].astype(jnp.bfloat16),
                          preferred_element_type=jnp.float32)
                  + b_ref[...])


def _pad2d(arr, rows, cols):
    r, c = arr.shape
    if r == rows and c == cols:
        return arr
    return jnp.pad(arr, ((0, rows - r), (0, cols - c)))


@jax.jit
def _gcn_forward(x, adj, w, b):
    n, f_in = x.shape
    f_out = w.shape[1]

    n_p = _round_up(n, _LANE)
    fin_p = _round_up(f_in, _LANE)
    fout_p = _round_up(f_out, _LANE)

    tm = 512
    while n_p % tm:
        tm //= 2

    adj_p = _pad2d(adj, n_p, n_p)
    x_p = _pad2d(x, n_p, fin_p)
    w_p = _pad2d(w, fin_p, fout_p)
    b_p = jnp.pad(b.astype(jnp.float32), (0, fout_p - f_out)).reshape(1, fout_p)

    cost = pl.CostEstimate(
        flops=2 * n_p * n_p * fin_p + 2 * n_p * fin_p * fout_p,
        transcendentals=0,
        bytes_accessed=(4 * n_p * n_p
                        + 2 * n_p * fin_p
                        + 2 * fin_p * fout_p
                        + 4 * fout_p
                        + 4 * n_p * fout_p))

    out = pl.pallas_call(
        _fused_kernel,
        out_shape=jax.ShapeDtypeStruct((n_p, fout_p), jnp.float32),
        grid=(n_p // tm,),
        in_specs=[
            pl.BlockSpec((tm, n_p), lambda i: (i, 0)),
            pl.BlockSpec((n_p, fin_p), lambda i: (0, 0),
                         pipeline_mode=pl.Buffered(1)),
            pl.BlockSpec((fin_p, fout_p), lambda i: (0, 0),
                         pipeline_mode=pl.Buffered(1)),
            pl.BlockSpec((1, fout_p), lambda i: (0, 0),
                         pipeline_mode=pl.Buffered(1)),
        ],
        out_specs=pl.BlockSpec((tm, fout_p), lambda i: (i, 0)),
        compiler_params=pltpu.CompilerParams(
            dimension_semantics=("parallel",),
            vmem_limit_bytes=56 * 1024 * 1024),
        cost_estimate=cost,
    )(adj_p, x_p, w_p, b_p)

    if n_p == n and fout_p == f_out:
        return out
    return out[:n, :f_out]


def kernel(x, adj, w, b):
    return _gcn_forward(x, adj, w, b)


# speedup vs baseline: 1.3565x; 1.0050x over previous
import functools

import jax
import jax.numpy as jnp
from jax.experimental import pallas as pl
from jax.experimental.pallas import tpu as pltpu

_LANE = 128


def _round_up(v, m):
    return ((v + m - 1) // m) * m


def _fused_kernel(a_ref, x_ref, w_ref, b_ref, o_ref):
    a16 = a_ref[...].astype(jnp.bfloat16)
    x16 = x_ref[...].astype(jnp.bfloat16)
    s = jnp.dot(a16, x16, preferred_element_type=jnp.float32)
    o_ref[...] = (jnp.dot(s.astype(jnp.bfloat16),
                          w_ref[...].astype(jnp.bfloat16),
                          preferred_element_type=jnp.float32)
                  + b_ref[...])


def _pad2d(arr, rows, cols):
    r, c = arr.shape
    if r == rows and c == cols:
        return arr
    return jnp.pad(arr, ((0, rows - r), (0, cols - c)))


@jax.jit
def _gcn_forward(x, adj, w, b):
    n, f_in = x.shape
    f_out = w.shape[1]

    n_p = _round_up(n, _LANE)
    fin_p = _round_up(f_in, _LANE)
    fout_p = _round_up(f_out, _LANE)

    tm = 1024
    while n_p % tm:
        tm //= 2

    adj_p = _pad2d(adj, n_p, n_p)
    x_p = _pad2d(x, n_p, fin_p)
    w_p = _pad2d(w, fin_p, fout_p)
    b_p = jnp.pad(b.astype(jnp.float32), (0, fout_p - f_out)).reshape(1, fout_p)

    cost = pl.CostEstimate(
        flops=2 * n_p * n_p * fin_p + 2 * n_p * fin_p * fout_p,
        transcendentals=0,
        bytes_accessed=(4 * n_p * n_p
                        + 2 * n_p * fin_p
                        + 2 * fin_p * fout_p
                        + 4 * fout_p
                        + 4 * n_p * fout_p))

    out = pl.pallas_call(
        _fused_kernel,
        out_shape=jax.ShapeDtypeStruct((n_p, fout_p), jnp.float32),
        grid=(n_p // tm,),
        in_specs=[
            pl.BlockSpec((tm, n_p), lambda i: (i, 0)),
            pl.BlockSpec((n_p, fin_p), lambda i: (0, 0),
                         pipeline_mode=pl.Buffered(1)),
            pl.BlockSpec((fin_p, fout_p), lambda i: (0, 0),
                         pipeline_mode=pl.Buffered(1)),
            pl.BlockSpec((1, fout_p), lambda i: (0, 0),
                         pipeline_mode=pl.Buffered(1)),
        ],
        out_specs=pl.BlockSpec((tm, fout_p), lambda i: (i, 0)),
        compiler_params=pltpu.CompilerParams(
            dimension_semantics=("parallel",),
            vmem_limit_bytes=56 * 1024 * 1024),
        cost_estimate=cost,
    )(adj_p, x_p, w_p, b_p)

    if n_p == n and fout_p == f_out:
        return out
    return out[:n, :f_out]


def kernel(x, adj, w, b):
    return _gcn_forward(x, adj, w, b)
